# baseline (speedup 1.0000x reference)
import functools

import jax
import jax.numpy as jnp
from jax import lax
from jax.experimental import pallas as pl
from jax.experimental.pallas import tpu as pltpu
from jax.experimental.pallas import tpu_sc as plsc

DIM = 128
KNBR = 16
NSLOT = KNBR + 1
BB = 2
NN = 10000
BN = BB * NN
NWORK = 32
ROWS_PER_W = 640
BNPAD = NWORK * ROWS_PER_W
SLAB = 16
SLABS_PER_W = ROWS_PER_W // SLAB
BM = 512
_HI = lax.Precision.HIGHEST


def _tc_body(x_ref, w1_ref, w2_ref, convt_ref, b1_ref, cb_ref, b2_ref,
             y_ref, z_ref, wstack, bias):
    @pl.when(pl.program_id(0) == 0)
    def _fold():
        w2 = w2_ref[...]
        for t in range(NSLOT):
            m = lax.dot(w2, convt_ref[t], precision=_HI)
            ws = m.T
            if t == 0:
                ws = ws + w1_ref[...].T
            wstack[t] = ws
        bias[...] = (b1_ref[...] + b2_ref[...]
                     + lax.dot(cb_ref[...], w2.T, precision=_HI))

    xb = x_ref[...]
    y_ref[...] = lax.dot(xb, wstack[0]) + bias[...]
    for t in range(KNBR):
        z_ref[t] = lax.dot(xb, wstack[t + 1])


_tc_call = pl.pallas_call(
    _tc_body,
    grid=(BNPAD // BM,),
    in_specs=[
        pl.BlockSpec((BM, DIM), lambda i: (i, 0)),
        pl.BlockSpec((DIM, DIM), lambda i: (0, 0)),
        pl.BlockSpec((DIM, DIM), lambda i: (0, 0)),
        pl.BlockSpec((NSLOT, DIM, DIM), lambda i: (0, 0, 0)),
        pl.BlockSpec((1, DIM), lambda i: (0, 0)),
        pl.BlockSpec((1, DIM), lambda i: (0, 0)),
        pl.BlockSpec((1, DIM), lambda i: (0, 0)),
    ],
    out_specs=[
        pl.BlockSpec((BM, DIM), lambda i: (i, 0)),
        pl.BlockSpec((KNBR, BM, DIM), lambda i: (0, i, 0)),
    ],
    out_shape=[
        jax.ShapeDtypeStruct((BNPAD, DIM), jnp.float32),
        jax.ShapeDtypeStruct((KNBR, BNPAD, DIM), jnp.float32),
    ],
    scratch_shapes=[
        pltpu.VMEM((NSLOT, DIM, DIM), jnp.float32),
        pltpu.VMEM((1, DIM), jnp.float32),
    ],
)


@functools.partial(
    pl.kernel,
    out_type=jax.ShapeDtypeStruct((BNPAD, DIM), jnp.float32),
    mesh=plsc.VectorSubcoreMesh(core_axis_name="c", subcore_axis_name="s"),
    scratch_types=[
        pltpu.VMEM((SLABS_PER_W, (KNBR * SLAB) // 128, 128), jnp.int32),
        pltpu.VMEM((2, KNBR, SLAB, DIM), jnp.float32),
        pltpu.VMEM((2, SLAB, DIM), jnp.float32),
        pltpu.VMEM((2, SLAB, DIM), jnp.float32),
        pltpu.SemaphoreType.DMA,
        pltpu.SemaphoreType.DMA,
        pltpu.SemaphoreType.DMA,
        pltpu.SemaphoreType.DMA,
        pltpu.SemaphoreType.DMA,
        pltpu.SemaphoreType.DMA,
    ],
)
def _sc_gather(y_hbm, z_hbm, idx_hbm, out_hbm, idxv, buf, ybuf, wbuf,
               sg0, sg1, sy0, sy1, sw0, sw1):
    semg = (sg0, sg1)
    semy = (sy0, sy1)
    semw = (sw0, sw1)
    wid = lax.axis_index("s") * 2 + lax.axis_index("c")
    base = wid * ROWS_PER_W
    pltpu.sync_copy(idx_hbm.at[wid], idxv)

    def islice(s, k):
        return idxv.at[s, k // 8, pl.ds((k % 8) * SLAB, SLAB)]

    def fire(s, b):
        r0 = base + s * SLAB
        pltpu.async_copy(y_hbm.at[pl.ds(r0, SLAB)], ybuf.at[b], semy[b])
        for k in range(KNBR):
            pltpu.async_copy(z_hbm.at[islice(s, k)], buf.at[b, k], semg[b])

    def drain(s, b):
        r0 = base + s * SLAB
        pltpu.make_async_copy(y_hbm.at[pl.ds(r0, SLAB)], ybuf.at[b],
                              semy[b]).wait()
        for k in range(KNBR):
            pltpu.make_async_copy(z_hbm.at[islice(s, k)], buf.at[b, k],
                                  semg[b]).wait()

    fire(0, 0)
    fire(1, 1)

    def pair_body(i, carry):
        for b in range(2):
            s = 2 * i + b
            r0 = base + s * SLAB
            drain(s, b)

            @pl.when(s >= 2)
            def _drain_prev_write():
                pltpu.make_async_copy(wbuf.at[b], out_hbm.at[pl.ds(r0, SLAB)],
                                      semw[b]).wait()

            def row_body(r, c2):
                for g in range(DIM // 16):
                    sl = pl.ds(g * 16, 16)
                    vals = [ybuf[b, r, sl]]
                    for k in range(KNBR):
                        vals.append(buf[b, k, r, sl])
                    while len(vals) > 1:
                        nxt = [vals[i] + vals[i + 1]
                               for i in range(0, len(vals) - 1, 2)]
                        if len(vals) % 2:
                            nxt.append(vals[-1])
                        vals = nxt
                    wbuf[b, r, sl] = vals[0]
                return c2

            lax.fori_loop(0, SLAB, row_body, 0)
            pltpu.async_copy(wbuf.at[b], out_hbm.at[pl.ds(r0, SLAB)], semw[b])

            @pl.when(s + 2 < SLABS_PER_W)
            def _fire_next():
                fire(s + 2, b)

        return carry

    lax.fori_loop(0, SLABS_PER_W // 2, pair_body, 0)
    for b in range(2):
        last = base + (SLABS_PER_W - 2 + b) * SLAB
        pltpu.make_async_copy(wbuf.at[b], out_hbm.at[pl.ds(last, SLAB)],
                              semw[b]).wait()


def kernel(x, adj_mat, W1_w, W1_b, conv_w, conv_b, W2_w, W2_b):
    x2 = x.reshape(BN, DIM)
    xpad = jnp.pad(x2, ((0, BNPAD - BN), (0, 0)))
    convt = jnp.transpose(conv_w, (2, 0, 1))
    b1 = W1_b.reshape(1, DIM)
    cb = conv_b.reshape(1, DIM)
    b2 = W2_b.reshape(1, DIM)
    y, z = _tc_call(xpad, W1_w, W2_w, convt, b1, cb, b2)
    zflat = z.reshape(KNBR * BNPAD, DIM)

    adj = adj_mat.astype(jnp.int32)
    flat = adj + (jnp.arange(BB, dtype=jnp.int32) * NN)[:, None, None]
    flat = flat.reshape(BN, KNBR)
    flat = jnp.pad(flat, ((0, BNPAD - BN), (0, 0)))
    flat = flat + (jnp.arange(KNBR, dtype=jnp.int32) * BNPAD)[None, :]
    idx4 = flat.reshape(NWORK, SLABS_PER_W, SLAB, KNBR).transpose(0, 1, 3, 2)
    idx4 = idx4.reshape(NWORK, SLABS_PER_W, (KNBR * SLAB) // 128, 128)

    out = _sc_gather(y, zflat, idx4)
    return out[:BN].reshape(BB, NN, DIM)


# speedup vs baseline: 1.0341x; 1.0341x over previous
import functools

import jax
import jax.numpy as jnp
from jax import lax
from jax.experimental import pallas as pl
from jax.experimental.pallas import tpu as pltpu
from jax.experimental.pallas import tpu_sc as plsc

DIM = 128
KNBR = 16
NSLOT = KNBR + 1
BB = 2
NN = 10000
BN = BB * NN
NWORK = 32
ROWS_PER_W = 640
BNPAD = NWORK * ROWS_PER_W
SLAB = 16
SLABS_PER_W = ROWS_PER_W // SLAB
BM = 2048
_HI = lax.Precision.HIGHEST


def _tc_body(x_ref, w1_ref, w2_ref, convt_ref, b1_ref, cb_ref, b2_ref,
             y_ref, z_ref, wstack, bias):
    @pl.when(pl.program_id(0) == 0)
    def _fold():
        w2 = w2_ref[...]
        for t in range(NSLOT):
            m = lax.dot(w2, convt_ref[t], precision=_HI)
            ws = m.T
            if t == 0:
                ws = ws + w1_ref[...].T
            wstack[t] = ws
        bias[...] = (b1_ref[...] + b2_ref[...]
                     + lax.dot(cb_ref[...], w2.T, precision=_HI))

    xb = x_ref[...]
    y_ref[...] = lax.dot(xb, wstack[0]) + bias[...]
    for t in range(KNBR):
        z_ref[t] = lax.dot(xb, wstack[t + 1])


_tc_call = pl.pallas_call(
    _tc_body,
    grid=(BNPAD // BM,),
    in_specs=[
        pl.BlockSpec((BM, DIM), lambda i: (i, 0)),
        pl.BlockSpec((DIM, DIM), lambda i: (0, 0)),
        pl.BlockSpec((DIM, DIM), lambda i: (0, 0)),
        pl.BlockSpec((NSLOT, DIM, DIM), lambda i: (0, 0, 0)),
        pl.BlockSpec((1, DIM), lambda i: (0, 0)),
        pl.BlockSpec((1, DIM), lambda i: (0, 0)),
        pl.BlockSpec((1, DIM), lambda i: (0, 0)),
    ],
    out_specs=[
        pl.BlockSpec((BM, DIM), lambda i: (i, 0)),
        pl.BlockSpec((KNBR, BM, DIM), lambda i: (0, i, 0)),
    ],
    out_shape=[
        jax.ShapeDtypeStruct((BNPAD, DIM), jnp.float32),
        jax.ShapeDtypeStruct((KNBR, BNPAD, DIM), jnp.float32),
    ],
    scratch_shapes=[
        pltpu.VMEM((NSLOT, DIM, DIM), jnp.float32),
        pltpu.VMEM((1, DIM), jnp.float32),
    ],
)


@functools.partial(
    pl.kernel,
    out_type=jax.ShapeDtypeStruct((BNPAD, DIM), jnp.float32),
    mesh=plsc.VectorSubcoreMesh(core_axis_name="c", subcore_axis_name="s"),
    scratch_types=[
        pltpu.VMEM((SLABS_PER_W, (KNBR * SLAB) // 128, 128), jnp.int32),
        pltpu.VMEM((2, KNBR, SLAB, DIM), jnp.float32),
        pltpu.VMEM((2, SLAB, DIM), jnp.float32),
        pltpu.VMEM((2, SLAB, DIM), jnp.float32),
        pltpu.SemaphoreType.DMA,
        pltpu.SemaphoreType.DMA,
        pltpu.SemaphoreType.DMA,
        pltpu.SemaphoreType.DMA,
        pltpu.SemaphoreType.DMA,
        pltpu.SemaphoreType.DMA,
    ],
)
def _sc_gather(y_hbm, z_hbm, idx_hbm, out_hbm, idxv, buf, ybuf, wbuf,
               sg0, sg1, sy0, sy1, sw0, sw1):
    semg = (sg0, sg1)
    semy = (sy0, sy1)
    semw = (sw0, sw1)
    wid = lax.axis_index("c") * 16 + lax.axis_index("s")
    base = wid * ROWS_PER_W
    pltpu.sync_copy(idx_hbm.at[wid], idxv)

    def islice(s, k):
        return idxv.at[s, k // 8, pl.ds((k % 8) * SLAB, SLAB)]

    def fire(s, b):
        r0 = base + s * SLAB
        pltpu.async_copy(y_hbm.at[pl.ds(r0, SLAB)], ybuf.at[b], semy[b])
        for k in range(KNBR):
            pltpu.async_copy(z_hbm.at[islice(s, k)], buf.at[b, k], semg[b])

    def drain(s, b):
        r0 = base + s * SLAB
        pltpu.make_async_copy(y_hbm.at[pl.ds(r0, SLAB)], ybuf.at[b],
                              semy[b]).wait()
        for k in range(KNBR):
            pltpu.make_async_copy(z_hbm.at[islice(s, k)], buf.at[b, k],
                                  semg[b]).wait()

    fire(0, 0)
    fire(1, 1)

    def pair_body(i, carry):
        for b in range(2):
            s = 2 * i + b
            r0 = base + s * SLAB
            drain(s, b)

            @pl.when(s >= 2)
            def _drain_prev_write():
                pltpu.make_async_copy(wbuf.at[b], out_hbm.at[pl.ds(r0, SLAB)],
                                      semw[b]).wait()

            def row_body(r, c2):
                for g in range(DIM // 16):
                    sl = pl.ds(g * 16, 16)
                    vals = [ybuf[b, r, sl]]
                    for k in range(KNBR):
                        vals.append(buf[b, k, r, sl])
                    while len(vals) > 1:
                        nxt = [vals[i] + vals[i + 1]
                               for i in range(0, len(vals) - 1, 2)]
                        if len(vals) % 2:
                            nxt.append(vals[-1])
                        vals = nxt
                    wbuf[b, r, sl] = vals[0]
                return c2

            lax.fori_loop(0, SLAB, row_body, 0)
            pltpu.async_copy(wbuf.at[b], out_hbm.at[pl.ds(r0, SLAB)], semw[b])

            @pl.when(s + 2 < SLABS_PER_W)
            def _fire_next():
                fire(s + 2, b)

        return carry

    lax.fori_loop(0, SLABS_PER_W // 2, pair_body, 0)
    for b in range(2):
        last = base + (SLABS_PER_W - 2 + b) * SLAB
        pltpu.make_async_copy(wbuf.at[b], out_hbm.at[pl.ds(last, SLAB)],
                              semw[b]).wait()


def kernel(x, adj_mat, W1_w, W1_b, conv_w, conv_b, W2_w, W2_b):
    x2 = x.reshape(BN, DIM)
    xpad = jnp.pad(x2, ((0, BNPAD - BN), (0, 0)))
    convt = jnp.transpose(conv_w, (2, 0, 1))
    b1 = W1_b.reshape(1, DIM)
    cb = conv_b.reshape(1, DIM)
    b2 = W2_b.reshape(1, DIM)
    y, z = _tc_call(xpad, W1_w, W2_w, convt, b1, cb, b2)
    zflat = z.reshape(KNBR * BNPAD, DIM)

    adj = adj_mat.astype(jnp.int32)
    flat = adj + (jnp.arange(BB, dtype=jnp.int32) * NN)[:, None, None]
    flat = flat.reshape(BN, KNBR)
    flat = jnp.pad(flat, ((0, BNPAD - BN), (0, 0)))
    flat = flat + (jnp.arange(KNBR, dtype=jnp.int32) * BNPAD)[None, :]
    idx4 = flat.reshape(NWORK, SLABS_PER_W, SLAB, KNBR).transpose(0, 1, 3, 2)
    idx4 = idx4.reshape(NWORK, SLABS_PER_W, (KNBR * SLAB) // 128, 128)

    out = _sc_gather(y, zflat, idx4)
    return out[:BN].reshape(BB, NN, DIM)


# speedup vs baseline: 1.2718x; 1.2299x over previous
import functools

import jax
import jax.numpy as jnp
from jax import lax
from jax.experimental import pallas as pl
from jax.experimental.pallas import tpu as pltpu
from jax.experimental.pallas import tpu_sc as plsc

DIM = 128
KNBR = 16
NSLOT = KNBR + 1
BB = 2
NN = 10000
BN = BB * NN
NWORK = 32
ROWS_PER_W = 640
BNPAD = NWORK * ROWS_PER_W
SLAB = 16
SLABS_PER_W = ROWS_PER_W // SLAB
BM = 2048
_HI = lax.Precision.HIGHEST


def _tc_body(x_ref, w1_ref, w2_ref, convt_ref, b1_ref, cb_ref, b2_ref,
             y_ref, z_ref, wstack, bias):
    @pl.when(pl.program_id(0) == 0)
    def _fold():
        w2 = w2_ref[...]
        for t in range(NSLOT):
            m = lax.dot(w2, convt_ref[t], precision=_HI)
            ws = m.T
            if t == 0:
                ws = ws + w1_ref[...].T
            wstack[t] = ws
        bias[...] = (b1_ref[...] + b2_ref[...]
                     + lax.dot(cb_ref[...], w2.T, precision=_HI))

    xb = x_ref[...]
    y_ref[...] = lax.dot(xb, wstack[0]) + bias[...]
    for t in range(KNBR):
        z_ref[t] = lax.dot(xb, wstack[t + 1])


_tc_call = pl.pallas_call(
    _tc_body,
    grid=(BNPAD // BM,),
    in_specs=[
        pl.BlockSpec((BM, DIM), lambda i: (i, 0)),
        pl.BlockSpec((DIM, DIM), lambda i: (0, 0)),
        pl.BlockSpec((DIM, DIM), lambda i: (0, 0)),
        pl.BlockSpec((NSLOT, DIM, DIM), lambda i: (0, 0, 0)),
        pl.BlockSpec((1, DIM), lambda i: (0, 0)),
        pl.BlockSpec((1, DIM), lambda i: (0, 0)),
        pl.BlockSpec((1, DIM), lambda i: (0, 0)),
    ],
    out_specs=[
        pl.BlockSpec((BM, DIM), lambda i: (i, 0)),
        pl.BlockSpec((KNBR, BM, DIM), lambda i: (0, i, 0)),
    ],
    out_shape=[
        jax.ShapeDtypeStruct((BNPAD, DIM), jnp.float32),
        jax.ShapeDtypeStruct((KNBR, BNPAD, DIM), jnp.float32),
    ],
    scratch_shapes=[
        pltpu.VMEM((NSLOT, DIM, DIM), jnp.float32),
        pltpu.VMEM((1, DIM), jnp.float32),
    ],
)


@functools.partial(
    pl.kernel,
    out_type=jax.ShapeDtypeStruct((BN, DIM), jnp.float32),
    mesh=plsc.VectorSubcoreMesh(core_axis_name="c", subcore_axis_name="s"),
    scratch_types=[
        pltpu.VMEM((SLABS_PER_W, (KNBR * SLAB) // 128, 128), jnp.int32),
        pltpu.VMEM((2, KNBR, SLAB, DIM), jnp.float32),
        pltpu.VMEM((2, SLAB, DIM), jnp.float32),
        pltpu.VMEM((2, SLAB, DIM), jnp.float32),
        pltpu.SemaphoreType.DMA,
        pltpu.SemaphoreType.DMA,
        pltpu.SemaphoreType.DMA,
        pltpu.SemaphoreType.DMA,
        pltpu.SemaphoreType.DMA,
        pltpu.SemaphoreType.DMA,
    ],
)
def _sc_gather(y_hbm, z_hbm, idx_hbm, out_hbm, idxv, buf, ybuf, wbuf,
               sg0, sg1, sy0, sy1, sw0, sw1):
    semg = (sg0, sg1)
    semy = (sy0, sy1)
    semw = (sw0, sw1)
    wid = lax.axis_index("c") * 16 + lax.axis_index("s")
    base = wid * ROWS_PER_W
    pltpu.sync_copy(idx_hbm.at[wid], idxv)

    def islice(s, k):
        return idxv.at[s, k // 8, pl.ds((k % 8) * SLAB, SLAB)]

    def fire(s, b):
        r0 = base + s * SLAB
        pltpu.async_copy(y_hbm.at[pl.ds(r0, SLAB)], ybuf.at[b], semy[b])
        for k in range(KNBR):
            pltpu.async_copy(z_hbm.at[islice(s, k)], buf.at[b, k], semg[b])

    def drain(s, b):
        r0 = base + s * SLAB
        pltpu.make_async_copy(y_hbm.at[pl.ds(r0, SLAB)], ybuf.at[b],
                              semy[b]).wait()
        for k in range(KNBR):
            pltpu.make_async_copy(z_hbm.at[islice(s, k)], buf.at[b, k],
                                  semg[b]).wait()

    fire(0, 0)
    fire(1, 1)

    def pair_body(i, carry):
        for b in range(2):
            s = 2 * i + b
            r0 = base + s * SLAB

            @pl.when(r0 < BN)
            def _slab():
                drain(s, b)

                @pl.when(s >= 2)
                def _drain_prev_write():
                    pltpu.make_async_copy(wbuf.at[b],
                                          out_hbm.at[pl.ds(r0, SLAB)],
                                          semw[b]).wait()

                def row_body(r, c2):
                    for g in range(DIM // 16):
                        sl = pl.ds(g * 16, 16)
                        vals = [ybuf[b, r, sl]]
                        for k in range(KNBR):
                            vals.append(buf[b, k, r, sl])
                        while len(vals) > 1:
                            nxt = [vals[i] + vals[i + 1]
                                   for i in range(0, len(vals) - 1, 2)]
                            if len(vals) % 2:
                                nxt.append(vals[-1])
                            vals = nxt
                        wbuf[b, r, sl] = vals[0]
                    return c2

                lax.fori_loop(0, SLAB, row_body, 0)
                pltpu.async_copy(wbuf.at[b], out_hbm.at[pl.ds(r0, SLAB)],
                                 semw[b])

                @pl.when((s + 2 < SLABS_PER_W)
                         & (r0 + 2 * SLAB < BN))
                def _fire_next():
                    fire(s + 2, b)

        return carry

    lax.fori_loop(0, SLABS_PER_W // 2, pair_body, 0)
    rows_real = lax.min(BN - base, ROWS_PER_W)
    for b in range(2):
        last = base + rows_real - (2 - b) * SLAB
        pltpu.make_async_copy(wbuf.at[b], out_hbm.at[pl.ds(last, SLAB)],
                              semw[b]).wait()


def kernel(x, adj_mat, W1_w, W1_b, conv_w, conv_b, W2_w, W2_b):
    x2 = x.reshape(BN, DIM)
    xpad = jnp.pad(x2, ((0, BNPAD - BN), (0, 0)))
    convt = jnp.transpose(conv_w, (2, 0, 1))
    b1 = W1_b.reshape(1, DIM)
    cb = conv_b.reshape(1, DIM)
    b2 = W2_b.reshape(1, DIM)
    y, z = _tc_call(xpad, W1_w, W2_w, convt, b1, cb, b2)
    zflat = z.reshape(KNBR * BNPAD, DIM)

    adj = adj_mat.astype(jnp.int32)
    flat = adj + (jnp.arange(BB, dtype=jnp.int32) * NN)[:, None, None]
    flat = flat.reshape(BN, KNBR)
    flat = jnp.pad(flat, ((0, BNPAD - BN), (0, 0)))
    flat = flat + (jnp.arange(KNBR, dtype=jnp.int32) * BNPAD)[None, :]
    idx4 = flat.reshape(NWORK, SLABS_PER_W, SLAB, KNBR).transpose(0, 1, 3, 2)
    idx4 = idx4.reshape(NWORK, SLABS_PER_W, (KNBR * SLAB) // 128, 128)

    out = _sc_gather(y, zflat, idx4)
    return out.reshape(BB, NN, DIM)


# speedup vs baseline: 1.3483x; 1.0601x over previous
import functools

import jax
import jax.numpy as jnp
from jax import lax
from jax.experimental import pallas as pl
from jax.experimental.pallas import tpu as pltpu
from jax.experimental.pallas import tpu_sc as plsc

DIM = 128
KNBR = 16
NSLOT = KNBR + 1
BB = 2
NN = 10000
BN = BB * NN
NWORK = 32
ROWS_PER_W = 640
BNPAD = NWORK * ROWS_PER_W
SLAB = 16
SLABS_PER_W = ROWS_PER_W // SLAB
BM = 2000
_HI = lax.Precision.HIGHEST


def _tc_body(x_ref, w1_ref, w2_ref, convt_ref, b1_ref, cb_ref, b2_ref,
             y_ref, z_ref, wstack, bias):
    @pl.when(pl.program_id(0) == 0)
    def _fold():
        w2 = w2_ref[...]
        for t in range(NSLOT):
            m = lax.dot(w2, convt_ref[t], precision=_HI)
            ws = m.T
            if t == 0:
                ws = ws + w1_ref[...].T
            wstack[t] = ws
        bias[...] = (b1_ref[...] + b2_ref[...]
                     + lax.dot(cb_ref[...], w2.T, precision=_HI))

    xb = x_ref[...]
    y_ref[...] = lax.dot(xb, wstack[0]) + bias[...]
    for t in range(KNBR):
        z_ref[t] = lax.dot(xb, wstack[t + 1])


_tc_call = pl.pallas_call(
    _tc_body,
    grid=(BN // BM,),
    in_specs=[
        pl.BlockSpec((BM, DIM), lambda i: (i, 0)),
        pl.BlockSpec((DIM, DIM), lambda i: (0, 0)),
        pl.BlockSpec((DIM, DIM), lambda i: (0, 0)),
        pl.BlockSpec((NSLOT, DIM, DIM), lambda i: (0, 0, 0)),
        pl.BlockSpec((1, DIM), lambda i: (0, 0)),
        pl.BlockSpec((1, DIM), lambda i: (0, 0)),
        pl.BlockSpec((1, DIM), lambda i: (0, 0)),
    ],
    out_specs=[
        pl.BlockSpec((BM, DIM), lambda i: (i, 0)),
        pl.BlockSpec((KNBR, BM, DIM), lambda i: (0, i, 0)),
    ],
    out_shape=[
        jax.ShapeDtypeStruct((BN, DIM), jnp.float32),
        jax.ShapeDtypeStruct((KNBR, BN, DIM), jnp.float32),
    ],
    scratch_shapes=[
        pltpu.VMEM((NSLOT, DIM, DIM), jnp.float32),
        pltpu.VMEM((1, DIM), jnp.float32),
    ],
)


@functools.partial(
    pl.kernel,
    out_type=jax.ShapeDtypeStruct((BN, DIM), jnp.float32),
    mesh=plsc.VectorSubcoreMesh(core_axis_name="c", subcore_axis_name="s"),
    scratch_types=[
        pltpu.VMEM((SLABS_PER_W, (KNBR * SLAB) // 128, 128), jnp.int32),
        pltpu.VMEM((2, KNBR, SLAB, DIM), jnp.float32),
        pltpu.VMEM((2, SLAB, DIM), jnp.float32),
        pltpu.VMEM((2, SLAB, DIM), jnp.float32),
        pltpu.SemaphoreType.DMA,
        pltpu.SemaphoreType.DMA,
        pltpu.SemaphoreType.DMA,
        pltpu.SemaphoreType.DMA,
        pltpu.SemaphoreType.DMA,
        pltpu.SemaphoreType.DMA,
    ],
)
def _sc_gather(y_hbm, z_hbm, idx_hbm, out_hbm, idxv, buf, ybuf, wbuf,
               sg0, sg1, sy0, sy1, sw0, sw1):
    semg = (sg0, sg1)
    semy = (sy0, sy1)
    semw = (sw0, sw1)
    wid = lax.axis_index("c") * 16 + lax.axis_index("s")
    base = wid * ROWS_PER_W
    pltpu.sync_copy(idx_hbm.at[wid], idxv)

    def islice(s, k):
        return idxv.at[s, k // 8, pl.ds((k % 8) * SLAB, SLAB)]

    def fire(s, b):
        r0 = base + s * SLAB
        pltpu.async_copy(y_hbm.at[pl.ds(r0, SLAB)], ybuf.at[b], semy[b])
        for k in range(KNBR):
            pltpu.async_copy(z_hbm.at[islice(s, k)], buf.at[b, k], semg[b])

    def drain(s, b):
        r0 = base + s * SLAB
        pltpu.make_async_copy(y_hbm.at[pl.ds(r0, SLAB)], ybuf.at[b],
                              semy[b]).wait()
        for k in range(KNBR):
            pltpu.make_async_copy(z_hbm.at[islice(s, k)], buf.at[b, k],
                                  semg[b]).wait()

    fire(0, 0)
    fire(1, 1)

    def pair_body(i, carry):
        for b in range(2):
            s = 2 * i + b
            r0 = base + s * SLAB

            @pl.when(r0 < BN)
            def _slab():
                drain(s, b)

                @pl.when(s >= 2)
                def _drain_prev_write():
                    pltpu.make_async_copy(wbuf.at[b],
                                          out_hbm.at[pl.ds(r0, SLAB)],
                                          semw[b]).wait()

                def row_body(r, c2):
                    for g in range(DIM // 16):
                        sl = pl.ds(g * 16, 16)
                        vals = [ybuf[b, r, sl]]
                        for k in range(KNBR):
                            vals.append(buf[b, k, r, sl])
                        while len(vals) > 1:
                            nxt = [vals[i] + vals[i + 1]
                                   for i in range(0, len(vals) - 1, 2)]
                            if len(vals) % 2:
                                nxt.append(vals[-1])
                            vals = nxt
                        wbuf[b, r, sl] = vals[0]
                    return c2

                lax.fori_loop(0, SLAB, row_body, 0)
                pltpu.async_copy(wbuf.at[b], out_hbm.at[pl.ds(r0, SLAB)],
                                 semw[b])

                @pl.when((s + 2 < SLABS_PER_W)
                         & (r0 + 2 * SLAB < BN))
                def _fire_next():
                    fire(s + 2, b)

        return carry

    lax.fori_loop(0, SLABS_PER_W // 2, pair_body, 0)
    rows_real = lax.min(BN - base, ROWS_PER_W)
    for b in range(2):
        last = base + rows_real - (2 - b) * SLAB
        pltpu.make_async_copy(wbuf.at[b], out_hbm.at[pl.ds(last, SLAB)],
                              semw[b]).wait()


def kernel(x, adj_mat, W1_w, W1_b, conv_w, conv_b, W2_w, W2_b):
    x2 = x.reshape(BN, DIM)
    convt = jnp.transpose(conv_w, (2, 0, 1))
    b1 = W1_b.reshape(1, DIM)
    cb = conv_b.reshape(1, DIM)
    b2 = W2_b.reshape(1, DIM)
    y, z = _tc_call(x2, W1_w, W2_w, convt, b1, cb, b2)
    zflat = z.reshape(KNBR * BN, DIM)

    adj = adj_mat.astype(jnp.int32)
    flat = adj + (jnp.arange(BB, dtype=jnp.int32) * NN)[:, None, None]
    flat = flat.reshape(BN, KNBR)
    flat = jnp.pad(flat, ((0, BNPAD - BN), (0, 0)))
    flat = flat + (jnp.arange(KNBR, dtype=jnp.int32) * BN)[None, :]
    idx4 = flat.reshape(NWORK, SLABS_PER_W, SLAB, KNBR).transpose(0, 1, 3, 2)
    idx4 = idx4.reshape(NWORK, SLABS_PER_W, (KNBR * SLAB) // 128, 128)

    out = _sc_gather(y, zflat, idx4)
    return out.reshape(BB, NN, DIM)
